# baseline (speedup 1.0000x reference)
import functools

import jax
import jax.numpy as jnp
import numpy as np
from jax import lax
from jax.experimental import pallas as pl
from jax.experimental.pallas import tpu as pltpu

B = 16384
EMBED_DIM = 128
NUM_GENRES = 100


_CHUNK = 128


def _make_sc_gather():
    from jax.experimental.pallas import tpu_sc as plsc

    info = plsc.get_sparse_core_info()
    nc, ns = info.num_cores, info.num_subcores
    nw = nc * ns
    b_per_w = B // nw
    n_chunks = b_per_w // _CHUNK

    mesh = plsc.VectorSubcoreMesh(core_axis_name="c", subcore_axis_name="s")

    @functools.partial(
        pl.kernel,
        mesh=mesh,
        out_type=jax.ShapeDtypeStruct((B, EMBED_DIM), jnp.bfloat16),
        scratch_types=[
            pltpu.VMEM((n_chunks, _CHUNK), jnp.int32),
            pltpu.VMEM((b_per_w, EMBED_DIM), jnp.float32),
            pltpu.VMEM((b_per_w, EMBED_DIM), jnp.bfloat16),
            pltpu.SemaphoreType.DMA,
            pltpu.SemaphoreType.DMA,
        ],
        compiler_params=pltpu.CompilerParams(needs_layout_passes=False),
    )
    def gather_kernel(idx_hbm, table_hbm, out_hbm, idx_v, rows_v, rows_bf,
                      gsem, wsem):
        wid = lax.axis_index("s") * nc + lax.axis_index("c")
        base = wid * b_per_w
        pltpu.sync_copy(idx_hbm.at[wid], idx_v)
        for j in range(n_chunks):
            pltpu.async_copy(
                table_hbm.at[idx_v.at[j]],
                rows_v.at[pl.ds(j * _CHUNK, _CHUNK)],
                gsem,
            )
        for j in range(n_chunks):
            pltpu.make_async_copy(
                table_hbm.at[idx_v.at[j]],
                rows_v.at[pl.ds(j * _CHUNK, _CHUNK)],
                gsem,
            ).wait()

            @functools.partial(plsc.parallel_loop, 0, _CHUNK, unroll=16)
            def pack_row(r):
                row = j * _CHUNK + r
                for c in range(EMBED_DIM // 32):
                    a = rows_v[row, pl.ds(c * 32, 16)]
                    b = rows_v[row, pl.ds(c * 32 + 16, 16)]
                    ua = plsc.bitcast(a, jnp.uint32) + jnp.uint32(0x8000)
                    ub = plsc.bitcast(b, jnp.uint32) + jnp.uint32(0x8000)
                    packed = (ua >> jnp.uint32(16)) | (
                        ub & jnp.uint32(0xFFFF0000))
                    rows_bf[row, pl.ds(c * 32, 32)] = plsc.bitcast(
                        packed, jnp.bfloat16)
            pltpu.async_copy(
                rows_bf.at[pl.ds(j * _CHUNK, _CHUNK)],
                out_hbm.at[pl.ds(base + j * _CHUNK, _CHUNK)],
                wsem,
            )
        for j in range(n_chunks):
            pltpu.make_async_copy(
                rows_bf.at[pl.ds(j * _CHUNK, _CHUNK)],
                out_hbm.at[pl.ds(base + j * _CHUNK, _CHUNK)],
                wsem,
            ).wait()

    return gather_kernel


_PERM = np.empty(EMBED_DIM, dtype=np.int32)
for _j in range(EMBED_DIM):
    _c, _r = _j // 32, _j % 32
    _PERM[_j] = _c * 32 + (_r // 2 if _r % 2 == 0 else 16 + (_r - 1) // 2)


_BN = 8192


def _mlp_body(uv_ref, gv_ref, wp_ref, bp_ref, w1u_ref, w1g_ref, b1_ref,
              w2r_ref, b2_ref, out_ref):
    g = jnp.dot(gv_ref[...], wp_ref[...], preferred_element_type=jnp.float32)
    g = jnp.maximum(g + bp_ref[...], 0.0)
    h = jnp.dot(uv_ref[...], w1u_ref[...], preferred_element_type=jnp.float32)
    h = h + jnp.dot(g, w1g_ref[...], preferred_element_type=jnp.float32)
    h = jnp.maximum(h + b1_ref[...], 0.0)
    r = lax.dot_general(w2r_ref[...], h, (((1,), (1,)), ((), ())),
                        preferred_element_type=jnp.float32)
    out_ref[...] = r[0] + b2_ref[0, 0]


def _mlp_call(uv, gv, wp, bp, w1u, w1g, b1, w2r, b2):
    full = lambda shape: pl.BlockSpec(shape, lambda i: (0,) * len(shape))
    return pl.pallas_call(
        _mlp_body,
        grid=(B // _BN,),
        in_specs=[
            pl.BlockSpec((_BN, EMBED_DIM), lambda i: (i, 0)),
            pl.BlockSpec((_BN, NUM_GENRES), lambda i: (i, 0)),
            full(wp.shape),
            full(bp.shape),
            full(w1u.shape),
            full(w1g.shape),
            full(b1.shape),
            full(w2r.shape),
            full(b2.shape),
        ],
        out_specs=pl.BlockSpec((_BN,), lambda i: (i,)),
        out_shape=jax.ShapeDtypeStruct((B,), jnp.float32),
    )(uv, gv, wp, bp, w1u, w1g, b1, w2r, b2)


@jax.jit
def _run(user_ids, genre_vectors, emb_table, W_proj, b_proj, W1, b1, W2, b2):
    gather = _make_sc_gather()
    idx3d = user_ids.astype(jnp.int32).reshape(-1, B // (32 * _CHUNK), _CHUNK)
    uv = gather(idx3d, emb_table)
    w1u_perm = W1[:EMBED_DIM][_PERM].astype(jnp.bfloat16)
    return _mlp_call(
        uv,
        genre_vectors.astype(jnp.bfloat16),
        W_proj.astype(jnp.bfloat16),
        b_proj.reshape(1, EMBED_DIM),
        w1u_perm,
        W1[EMBED_DIM:],
        b1.reshape(1, 64),
        W2.reshape(1, 64),
        b2.reshape(1, 1),
    )


def kernel(user_ids, genre_vectors, emb_table, W_proj, b_proj, W1, b1, W2, b2):
    return _run(user_ids, genre_vectors, emb_table, W_proj, b_proj, W1, b1, W2,
                b2)


# speedup vs baseline: 1.0227x; 1.0227x over previous
import functools

import jax
import jax.numpy as jnp
from jax import lax
from jax.experimental import pallas as pl
from jax.experimental.pallas import tpu as pltpu

B = 16384
EMBED_DIM = 128
NUM_GENRES = 100


_CHUNK = 128


def _make_sc_gather():
    from jax.experimental.pallas import tpu_sc as plsc

    info = plsc.get_sparse_core_info()
    nc, ns = info.num_cores, info.num_subcores
    nw = nc * ns
    b_per_w = B // nw
    n_chunks = b_per_w // _CHUNK

    mesh = plsc.VectorSubcoreMesh(core_axis_name="c", subcore_axis_name="s")

    @functools.partial(
        pl.kernel,
        mesh=mesh,
        out_type=jax.ShapeDtypeStruct((B, EMBED_DIM), jnp.float32),
        scratch_types=[
            pltpu.VMEM((n_chunks, _CHUNK), jnp.int32),
            pltpu.VMEM((b_per_w, EMBED_DIM), jnp.float32),
            pltpu.SemaphoreType.DMA,
            pltpu.SemaphoreType.DMA,
        ],
    )
    def gather_kernel(idx_hbm, table_hbm, out_hbm, idx_v, rows_v, gsem, wsem):
        wid = lax.axis_index("s") * nc + lax.axis_index("c")
        base = wid * b_per_w
        pltpu.sync_copy(idx_hbm.at[wid], idx_v)
        for j in range(n_chunks):
            pltpu.async_copy(
                table_hbm.at[idx_v.at[j]],
                rows_v.at[pl.ds(j * _CHUNK, _CHUNK)],
                gsem,
            )
        for j in range(n_chunks):
            pltpu.make_async_copy(
                table_hbm.at[idx_v.at[j]],
                rows_v.at[pl.ds(j * _CHUNK, _CHUNK)],
                gsem,
            ).wait()
            pltpu.async_copy(
                rows_v.at[pl.ds(j * _CHUNK, _CHUNK)],
                out_hbm.at[pl.ds(base + j * _CHUNK, _CHUNK)],
                wsem,
            )
        for j in range(n_chunks):
            pltpu.make_async_copy(
                rows_v.at[pl.ds(j * _CHUNK, _CHUNK)],
                out_hbm.at[pl.ds(base + j * _CHUNK, _CHUNK)],
                wsem,
            ).wait()

    return gather_kernel


_BN = 4096


def _mlp_body(uv_ref, gv_ref, wp_ref, bp_ref, w1_ref, b1_ref, w2r_ref, b2_ref,
              out_ref):
    g = jnp.dot(gv_ref[...], wp_ref[...], preferred_element_type=jnp.float32)
    g = jnp.maximum(g + bp_ref[...], 0.0)
    h = jnp.dot(uv_ref[...], w1_ref[:EMBED_DIM, :],
                preferred_element_type=jnp.float32)
    h = h + jnp.dot(g, w1_ref[EMBED_DIM:, :], preferred_element_type=jnp.float32)
    h = jnp.maximum(h + b1_ref[...], 0.0)
    r = lax.dot_general(w2r_ref[...], h, (((1,), (1,)), ((), ())),
                        preferred_element_type=jnp.float32)
    out_ref[...] = r[0] + b2_ref[0, 0]


def _mlp_call(uv, gv, wp, bp, w1, b1, w2r, b2):
    full = lambda shape: pl.BlockSpec(shape, lambda i: (0,) * len(shape))
    return pl.pallas_call(
        _mlp_body,
        grid=(B // _BN,),
        in_specs=[
            pl.BlockSpec((_BN, EMBED_DIM), lambda i: (i, 0)),
            pl.BlockSpec((_BN, NUM_GENRES), lambda i: (i, 0)),
            full(wp.shape),
            full(bp.shape),
            full(w1.shape),
            full(b1.shape),
            full(w2r.shape),
            full(b2.shape),
        ],
        out_specs=pl.BlockSpec((_BN,), lambda i: (i,)),
        out_shape=jax.ShapeDtypeStruct((B,), jnp.float32),
    )(uv, gv, wp, bp, w1, b1, w2r, b2)


@jax.jit
def _run(user_ids, genre_vectors, emb_table, W_proj, b_proj, W1, b1, W2, b2):
    gather = _make_sc_gather()
    idx3d = user_ids.astype(jnp.int32).reshape(-1, B // (32 * _CHUNK), _CHUNK)
    uv = gather(idx3d, emb_table)
    return _mlp_call(
        uv,
        genre_vectors.astype(jnp.bfloat16),
        W_proj.astype(jnp.bfloat16),
        b_proj.reshape(1, EMBED_DIM),
        W1,
        b1.reshape(1, 64),
        W2.reshape(1, 64),
        b2.reshape(1, 1),
    )


def kernel(user_ids, genre_vectors, emb_table, W_proj, b_proj, W1, b1, W2, b2):
    return _run(user_ids, genre_vectors, emb_table, W_proj, b_proj, W1, b1, W2,
                b2)


# speedup vs baseline: 1.0228x; 1.0000x over previous
import functools

import jax
import jax.numpy as jnp
from jax import lax
from jax.experimental import pallas as pl
from jax.experimental.pallas import tpu as pltpu

B = 16384
EMBED_DIM = 128
NUM_GENRES = 100


_CHUNK = 128


def _make_sc_gather():
    from jax.experimental.pallas import tpu_sc as plsc

    info = plsc.get_sparse_core_info()
    nc, ns = info.num_cores, info.num_subcores
    nw = nc * ns
    b_per_w = B // nw
    n_chunks = b_per_w // _CHUNK

    mesh = plsc.VectorSubcoreMesh(core_axis_name="c", subcore_axis_name="s")

    @functools.partial(
        pl.kernel,
        mesh=mesh,
        out_type=jax.ShapeDtypeStruct((B, EMBED_DIM), jnp.float32),
        scratch_types=[
            pltpu.VMEM((n_chunks, _CHUNK), jnp.int32),
            pltpu.VMEM((b_per_w, EMBED_DIM), jnp.float32),
            pltpu.SemaphoreType.DMA,
            pltpu.SemaphoreType.DMA,
        ],
    )
    def gather_kernel(idx_hbm, table_hbm, out_hbm, idx_v, rows_v, gsem, wsem):
        wid = lax.axis_index("s") * nc + lax.axis_index("c")
        base = wid * b_per_w
        pltpu.sync_copy(idx_hbm.at[wid], idx_v)
        for j in range(n_chunks):
            pltpu.async_copy(
                table_hbm.at[idx_v.at[j]],
                rows_v.at[pl.ds(j * _CHUNK, _CHUNK)],
                gsem,
            )
        for j in range(n_chunks):
            pltpu.make_async_copy(
                table_hbm.at[idx_v.at[j]],
                rows_v.at[pl.ds(j * _CHUNK, _CHUNK)],
                gsem,
            ).wait()
            pltpu.async_copy(
                rows_v.at[pl.ds(j * _CHUNK, _CHUNK)],
                out_hbm.at[pl.ds(base + j * _CHUNK, _CHUNK)],
                wsem,
            )
        for j in range(n_chunks):
            pltpu.make_async_copy(
                rows_v.at[pl.ds(j * _CHUNK, _CHUNK)],
                out_hbm.at[pl.ds(base + j * _CHUNK, _CHUNK)],
                wsem,
            ).wait()

    return gather_kernel


_BN = 8192


def _mlp_body(uv_ref, gv_ref, wp_ref, bp_ref, w1_ref, b1_ref, w2r_ref, b2_ref,
              out_ref):
    g = jnp.dot(gv_ref[...], wp_ref[...], preferred_element_type=jnp.float32)
    g = jnp.maximum(g + bp_ref[...], 0.0)
    h = jnp.dot(uv_ref[...], w1_ref[:EMBED_DIM, :],
                preferred_element_type=jnp.float32)
    h = h + jnp.dot(g, w1_ref[EMBED_DIM:, :], preferred_element_type=jnp.float32)
    h = jnp.maximum(h + b1_ref[...], 0.0)
    r = lax.dot_general(w2r_ref[...], h, (((1,), (1,)), ((), ())),
                        preferred_element_type=jnp.float32)
    out_ref[...] = r[0] + b2_ref[0, 0]


def _mlp_call(uv, gv, wp, bp, w1, b1, w2r, b2):
    full = lambda shape: pl.BlockSpec(shape, lambda i: (0,) * len(shape))
    return pl.pallas_call(
        _mlp_body,
        grid=(B // _BN,),
        in_specs=[
            pl.BlockSpec((_BN, EMBED_DIM), lambda i: (i, 0)),
            pl.BlockSpec((_BN, NUM_GENRES), lambda i: (i, 0)),
            full(wp.shape),
            full(bp.shape),
            full(w1.shape),
            full(b1.shape),
            full(w2r.shape),
            full(b2.shape),
        ],
        out_specs=pl.BlockSpec((_BN,), lambda i: (i,)),
        out_shape=jax.ShapeDtypeStruct((B,), jnp.float32),
    )(uv, gv, wp, bp, w1, b1, w2r, b2)


@jax.jit
def _run(user_ids, genre_vectors, emb_table, W_proj, b_proj, W1, b1, W2, b2):
    gather = _make_sc_gather()
    idx3d = user_ids.astype(jnp.int32).reshape(-1, B // (32 * _CHUNK), _CHUNK)
    uv = gather(idx3d, emb_table)
    return _mlp_call(
        uv,
        genre_vectors.astype(jnp.bfloat16),
        W_proj.astype(jnp.bfloat16),
        b_proj.reshape(1, EMBED_DIM),
        W1,
        b1.reshape(1, 64),
        W2.reshape(1, 64),
        b2.reshape(1, 1),
    )


def kernel(user_ids, genre_vectors, emb_table, W_proj, b_proj, W1, b1, W2, b2):
    return _run(user_ids, genre_vectors, emb_table, W_proj, b_proj, W1, b1, W2,
                b2)
